# baseline (speedup 1.0000x reference)
import functools

import jax
import jax.numpy as jnp
from jax import lax
from jax.experimental import pallas as pl
from jax.experimental.pallas import tpu as pltpu
from jax.experimental.pallas import tpu_sc as plsc

_PREFIX_LEN = 5
_N_CLS_CTX = 4
_TOTAL_LEN = 77


def _make_sc_gather(V, D, B):
    info = plsc.get_sparse_core_info()
    NC, NS = info.num_cores, info.num_subcores
    NW = NC * NS
    assert B % (8 * NW) == 0
    b_per_w = B // NW
    mesh = plsc.VectorSubcoreMesh(core_axis_name="c", subcore_axis_name="s")

    @functools.partial(
        pl.kernel,
        mesh=mesh,
        out_type=jax.ShapeDtypeStruct((B, D), jnp.float32),
        scratch_types=[
            pltpu.VMEM((b_per_w,), jnp.int32),
            pltpu.VMEM((b_per_w, D), jnp.float32),
            pltpu.SemaphoreType.DMA,
        ],
    )
    def gather_kernel(table_hbm, idx_hbm, out_hbm, idx_v, rows_v, sem):
        wid = lax.axis_index("s") * NC + lax.axis_index("c")
        base = wid * b_per_w
        pltpu.sync_copy(idx_hbm.at[pl.ds(base, b_per_w)], idx_v)
        pltpu.async_copy(table_hbm.at[idx_v], rows_v, sem).wait()
        pltpu.sync_copy(rows_v, out_hbm.at[pl.ds(base, b_per_w)])

    return gather_kernel


def _asm_body(cls_ref, pre_ref, suf_ref, out_ref, *, bb, pw, cw, tw):
    out_ref[:, 0:pw] = jnp.broadcast_to(pre_ref[...], (bb, pw))
    out_ref[:, pw:pw + cw] = cls_ref[...]
    out_ref[:, pw + cw:tw] = jnp.broadcast_to(suf_ref[...], (bb, tw - pw - cw))


def _tc_assemble(gathered, pre2, suf2, *, bb):
    B, cw = gathered.shape
    pw = pre2.shape[1]
    sw = suf2.shape[1]
    tw = pw + cw + sw
    body = functools.partial(_asm_body, bb=bb, pw=pw, cw=cw, tw=tw)
    return pl.pallas_call(
        body,
        grid=(B // bb,),
        in_specs=[
            pl.BlockSpec((bb, cw), lambda i: (i, 0)),
            pl.BlockSpec((1, pw), lambda i: (0, 0)),
            pl.BlockSpec((1, sw), lambda i: (0, 0)),
        ],
        out_specs=pl.BlockSpec((bb, tw), lambda i: (i, 0)),
        out_shape=jax.ShapeDtypeStruct((B, tw), jnp.float32),
    )(gathered, pre2, suf2)


@jax.jit
def kernel(label, cls_ctx, prefix_base, suffix_base):
    B = label.shape[0]
    V, C, D = cls_ctx.shape
    table2 = cls_ctx.reshape(V, C * D)
    idx = label.astype(jnp.int32)
    gathered = _make_sc_gather(V, C * D, B)(table2, idx)
    pre2 = prefix_base.reshape(1, _PREFIX_LEN * D)
    suf2 = suffix_base.reshape(1, (_TOTAL_LEN - _PREFIX_LEN - _N_CLS_CTX) * D)
    out2 = _tc_assemble(gathered, pre2, suf2, bb=16)
    return out2.reshape(B, _TOTAL_LEN, D)


# speedup vs baseline: 9.5271x; 9.5271x over previous
import functools

import jax
import jax.numpy as jnp
from jax import lax
from jax.experimental import pallas as pl
from jax.experimental.pallas import tpu as pltpu
from jax.experimental.pallas import tpu_sc as plsc

_PRE = 5
_NCLS = 4
_TOT = 77
_SUF = _TOT - _PRE - _NCLS
_L = 128
_NCHUNK = 4


def _make_sc_gather(V, B):
    info = plsc.get_sparse_core_info()
    NC, NS, L = info.num_cores, info.num_subcores, info.num_lanes
    NW = NC * NS
    assert B % (8 * NW) == 0 and L == 16
    bw = B // NW
    n_r = _NCLS * _NCHUNK

    mesh = plsc.VectorSubcoreMesh(core_axis_name="c", subcore_axis_name="s")

    @functools.partial(
        pl.kernel,
        mesh=mesh,
        out_type=jax.ShapeDtypeStruct((_NCLS, _NCHUNK, B, _L), jnp.float32),
        scratch_types=[
            pltpu.VMEM((bw,), jnp.int32),
            pltpu.VMEM((bw,), jnp.int32),
            pltpu.VMEM((n_r, bw), jnp.int32),
            pltpu.VMEM((n_r, bw, _L), jnp.float32),
            pltpu.SemaphoreType.DMA,
        ],
    )
    def gather_kernel(table_hbm, idx_hbm, out_hbm, lbl_v, lbl16_v, idx_v,
                      rows_v, sem):
        wid = lax.axis_index("s") * NC + lax.axis_index("c")
        base = wid * bw
        pltpu.sync_copy(idx_hbm.at[pl.ds(base, bw)], lbl_v)
        for j0 in range(0, bw, L):
            lbl16_v[pl.ds(j0, L)] = lbl_v[pl.ds(j0, L)] * (_NCLS * _NCHUNK)
        for t in range(_NCLS):
            for c in range(_NCHUNK):
                r = t * _NCHUNK + c
                for j0 in range(0, bw, L):
                    idx_v[r, pl.ds(j0, L)] = (
                        lbl16_v[pl.ds(j0, L)] + (c * _NCLS + t)
                    )
        copies = []
        for r in range(n_r):
            copies.append(
                pltpu.async_copy(table_hbm.at[idx_v.at[r]], rows_v.at[r], sem)
            )
        for d in copies:
            d.wait()
        for t in range(_NCLS):
            for c in range(_NCHUNK):
                r = t * _NCHUNK + c
                pltpu.sync_copy(
                    rows_v.at[r], out_hbm.at[t, c, pl.ds(base, bw), :]
                )

    return gather_kernel


def _asm_body(cls_ref, pre_ref, suf_ref, out_ref, *, B, D):
    t = pl.program_id(0)

    @pl.when(t < _PRE)
    def _():
        out_ref[0] = jnp.broadcast_to(pre_ref[0], (B, D))

    @pl.when(jnp.logical_and(t >= _PRE, t < _PRE + _NCLS))
    def _():
        for c in range(_NCHUNK):
            out_ref[0, :, c * _L:(c + 1) * _L] = cls_ref[0, c]

    @pl.when(t >= _PRE + _NCLS)
    def _():
        out_ref[0] = jnp.broadcast_to(suf_ref[0], (B, D))


def _tc_assemble(gathered, pre3, suf3):
    B = gathered.shape[2]
    D = _NCHUNK * _L
    body = functools.partial(_asm_body, B=B, D=D)
    return pl.pallas_call(
        body,
        grid=(_TOT,),
        in_specs=[
            pl.BlockSpec(
                (1, _NCHUNK, B, _L),
                lambda t: (jnp.clip(t - _PRE, 0, _NCLS - 1), 0, 0, 0),
            ),
            pl.BlockSpec((1, 1, D), lambda t: (jnp.clip(t, 0, _PRE - 1), 0, 0)),
            pl.BlockSpec(
                (1, 1, D),
                lambda t: (jnp.clip(t - _PRE - _NCLS, 0, _SUF - 1), 0, 0),
            ),
        ],
        out_specs=pl.BlockSpec((1, B, D), lambda t: (t, 0, 0)),
        out_shape=jax.ShapeDtypeStruct((_TOT, B, D), jnp.float32),
    )(gathered, pre3, suf3)


@jax.jit
def kernel(label, cls_ctx, prefix_base, suffix_base):
    B = label.shape[0]
    V, C, D = cls_ctx.shape
    table_sc = (
        cls_ctx.reshape(V, C, D // _L, _L)
        .transpose(0, 2, 1, 3)
        .reshape(V * C * (D // _L), _L)
    )
    idx = label.astype(jnp.int32)
    gathered = _make_sc_gather(V, B)(table_sc, idx)
    pre3 = prefix_base.reshape(_PRE, 1, D)
    suf3 = suffix_base.reshape(_SUF, 1, D)
    out_tm = _tc_assemble(gathered, pre3, suf3)
    return out_tm.transpose(1, 0, 2)


# speedup vs baseline: 10.3323x; 1.0845x over previous
import functools

import jax
import jax.numpy as jnp
from jax import lax
from jax.experimental import pallas as pl
from jax.experimental.pallas import tpu as pltpu
from jax.experimental.pallas import tpu_sc as plsc

_PRE = 5
_NCLS = 4
_TOT = 77
_SUF = _TOT - _PRE - _NCLS
_L = 128
_NCHUNK = 4


def _make_sc_gather(V, B):
    info = plsc.get_sparse_core_info()
    NC, NS, L = info.num_cores, info.num_subcores, info.num_lanes
    NW = NC * NS
    assert B % (8 * NW) == 0 and L == 16
    bw = B // NW
    n_r = _NCLS * _NCHUNK

    mesh = plsc.VectorSubcoreMesh(core_axis_name="c", subcore_axis_name="s")

    @functools.partial(
        pl.kernel,
        mesh=mesh,
        out_type=jax.ShapeDtypeStruct((_NCLS, _NCHUNK, B, _L), jnp.float32),
        scratch_types=[
            pltpu.VMEM((bw,), jnp.int32),
            pltpu.VMEM((bw,), jnp.int32),
            pltpu.VMEM((n_r, bw), jnp.int32),
            pltpu.VMEM((n_r, bw, _L), jnp.float32),
            pltpu.SemaphoreType.DMA,
        ],
    )
    def gather_kernel(table_hbm, idx_hbm, out_hbm, lbl_v, lbl16_v, idx_v,
                      rows_v, sem):
        wid = lax.axis_index("s") * NC + lax.axis_index("c")
        base = wid * bw
        pltpu.sync_copy(idx_hbm.at[pl.ds(base, bw)], lbl_v)
        for j0 in range(0, bw, L):
            lbl16_v[pl.ds(j0, L)] = lbl_v[pl.ds(j0, L)] * (_NCLS * _NCHUNK)
        for t in range(_NCLS):
            for c in range(_NCHUNK):
                r = t * _NCHUNK + c
                for j0 in range(0, bw, L):
                    idx_v[r, pl.ds(j0, L)] = (
                        lbl16_v[pl.ds(j0, L)] + (c * _NCLS + t)
                    )
        copies = []
        for r in range(n_r):
            copies.append(
                pltpu.async_copy(table_hbm.at[idx_v.at[r]], rows_v.at[r], sem)
            )
        for d in copies:
            d.wait()
        for t in range(_NCLS):
            for c in range(_NCHUNK):
                r = t * _NCHUNK + c
                pltpu.sync_copy(
                    rows_v.at[r], out_hbm.at[t, c, pl.ds(base, bw), :]
                )

    return gather_kernel


def _presuf_body(pre_ref, suf_ref, out_ref, *, B, D):
    t = pl.program_id(0)

    @pl.when(t < _PRE)
    def _():
        out_ref[0] = jnp.broadcast_to(pre_ref[0], (B, D))

    @pl.when(t >= _PRE)
    def _():
        out_ref[0] = jnp.broadcast_to(suf_ref[0], (B, D))


def _tc_presuf(pre3, suf3, B):
    D = _NCHUNK * _L
    body = functools.partial(_presuf_body, B=B, D=D)
    return pl.pallas_call(
        body,
        grid=(_TOT - _NCLS,),
        in_specs=[
            pl.BlockSpec((1, 1, D), lambda t: (jnp.clip(t, 0, _PRE - 1), 0, 0)),
            pl.BlockSpec(
                (1, 1, D), lambda t: (jnp.clip(t - _PRE, 0, _SUF - 1), 0, 0)
            ),
        ],
        out_specs=pl.BlockSpec(
            (1, B, D),
            lambda t: (jnp.where(t < _PRE, t, t + _NCLS), 0, 0),
        ),
        out_shape=jax.ShapeDtypeStruct((_TOT, B, D), jnp.float32),
    )(pre3, suf3)


def _cls_body(al_ref, cls_ref, out_ref):
    for c in range(_NCHUNK):
        out_ref[0, :, c * _L:(c + 1) * _L] = cls_ref[0, c]


def _tc_fill_cls(out_tm, gathered):
    B = gathered.shape[2]
    D = _NCHUNK * _L
    return pl.pallas_call(
        _cls_body,
        grid=(_NCLS,),
        in_specs=[
            pl.BlockSpec(memory_space=pl.ANY),
            pl.BlockSpec((1, _NCHUNK, B, _L), lambda t: (t, 0, 0, 0)),
        ],
        out_specs=pl.BlockSpec((1, B, D), lambda t: (t + _PRE, 0, 0)),
        out_shape=jax.ShapeDtypeStruct((_TOT, B, D), jnp.float32),
        input_output_aliases={0: 0},
    )(out_tm, gathered)


@jax.jit
def kernel(label, cls_ctx, prefix_base, suffix_base):
    B = label.shape[0]
    V, C, D = cls_ctx.shape
    table_sc = (
        cls_ctx.reshape(V, C, D // _L, _L)
        .transpose(0, 2, 1, 3)
        .reshape(V * C * (D // _L), _L)
    )
    idx = label.astype(jnp.int32)
    gathered = _make_sc_gather(V, B)(table_sc, idx)
    pre3 = prefix_base.reshape(_PRE, 1, D)
    suf3 = suffix_base.reshape(_SUF, 1, D)
    out_tm = _tc_presuf(pre3, suf3, B)
    out_tm = _tc_fill_cls(out_tm, gathered)
    return out_tm.transpose(1, 0, 2)
